# Initial kernel scaffold; baseline (speedup 1.0000x reference)
import jax
import jax.numpy as jnp
from jax.experimental import pallas as pl


def kernel(x, edge_index, W1, b1, W_ih, W_hh, b_ih, b_hh, W_fc, b_fc):
    raise NotImplementedError("write your pallas kernel here")


# speedup vs baseline: 7.4438x; 7.4438x over previous
import functools

import jax
import jax.numpy as jnp
from jax import lax
from jax.experimental import pallas as pl
from jax.experimental.pallas import tpu as pltpu
from jax.experimental.pallas import tpu_sc as plsc

_N = 10000
_E = 320000
_D = 128
_H = 128
_K = 5000
_SORTN = 16384

_NCORES = 2
_NSUB = 16
_EPC = _E // _NCORES
_EPW = _E // (_NCORES * _NSUB)
_CH = 80
_NCH = _EPW // _CH
_NP = 10240
_NPH = _NP // 2

_HIGH = jax.lax.Precision.HIGHEST


def _sc_degree(dstp0, dstp1):
    mesh = plsc.VectorSubcoreMesh(core_axis_name="c", subcore_axis_name="s")
    rps = _NPH // _NSUB

    @functools.partial(
        pl.kernel,
        mesh=mesh,
        out_type=jax.ShapeDtypeStruct((_NCORES, _NP, _D), jnp.float32),
        scratch_types=[
            pltpu.VMEM((_CH,), jnp.int32),
            pltpu.VMEM((_CH, _D), jnp.float32),
            pltpu.VMEM((rps, _D), jnp.float32),
            pltpu.VMEM_SHARED((_NPH + 16, _D), jnp.float32),
        ],
    )
    def k(dstp0_hbm, dstp1_hbm, out_hbm, idx_v, ones_v, zbuf_v, acc_sh):
        c = lax.axis_index("c")
        s = lax.axis_index("s")
        zero16 = jnp.zeros((16,), jnp.float32)
        ones16 = jnp.ones((16,), jnp.float32)

        def fill_ones(i, carry):
            for j in range(_D // 16):
                ones_v[i, pl.ds(j * 16, 16)] = ones16
            return carry

        lax.fori_loop(0, _CH, fill_ones, 0)

        def fill_zero(i, carry):
            for j in range(_D // 16):
                zbuf_v[i, pl.ds(j * 16, 16)] = zero16
            return carry

        lax.fori_loop(0, rps, fill_zero, 0)

        ebase = c * _EPC + s * _EPW

        for off, dst_hbm in ((0, dstp0_hbm), (_NPH, dstp1_hbm)):
            pltpu.sync_copy(zbuf_v, acc_sh.at[pl.ds(s * rps, rps)])
            plsc.subcore_barrier()

            def chunk(i, carry):
                b = pl.multiple_of(ebase + i * _CH, 8)
                pltpu.sync_copy(dst_hbm.at[pl.ds(b, _CH)], idx_v)
                pltpu.sync_copy(ones_v, acc_sh.at[idx_v], add=True)
                return carry

            lax.fori_loop(0, _NCH, chunk, 0)

            plsc.subcore_barrier()
            pltpu.sync_copy(acc_sh.at[pl.ds(s * rps, rps)],
                            out_hbm.at[c, pl.ds(off + s * rps, rps)])

    return k(dstp0, dstp1)


def _sc_scatter_rows(rows, src, dstp0, dstp1):
    mesh = plsc.VectorSubcoreMesh(core_axis_name="c", subcore_axis_name="s")
    rps = _NPH // _NSUB

    @functools.partial(
        pl.kernel,
        mesh=mesh,
        out_type=jax.ShapeDtypeStruct((_NCORES, _NP, _D), jnp.float32),
        scratch_types=[
            pltpu.VMEM((_CH,), jnp.int32),
            pltpu.VMEM((_CH,), jnp.int32),
            pltpu.VMEM((_CH, _D), jnp.float32),
            pltpu.VMEM((rps, _D), jnp.float32),
            pltpu.VMEM_SHARED((_NPH + 16, _D), jnp.float32),
            pltpu.SemaphoreType.DMA,
        ],
    )
    def k(rows_hbm, src_hbm, dstp0_hbm, dstp1_hbm, out_hbm,
          isrc_v, idst_v, rows_v, zbuf_v, acc_sh, sem):
        c = lax.axis_index("c")
        s = lax.axis_index("s")
        zero16 = jnp.zeros((16,), jnp.float32)

        def fill_zero(i, carry):
            for j in range(_D // 16):
                zbuf_v[i, pl.ds(j * 16, 16)] = zero16
            return carry

        lax.fori_loop(0, rps, fill_zero, 0)

        ebase = c * _EPC + s * _EPW

        for off, dst_hbm in ((0, dstp0_hbm), (_NPH, dstp1_hbm)):
            pltpu.sync_copy(zbuf_v, acc_sh.at[pl.ds(s * rps, rps)])
            plsc.subcore_barrier()

            def chunk(i, carry):
                b = pl.multiple_of(ebase + i * _CH, 8)
                pltpu.sync_copy(src_hbm.at[pl.ds(b, _CH)], isrc_v)
                pltpu.sync_copy(dst_hbm.at[pl.ds(b, _CH)], idst_v)
                pltpu.async_copy(rows_hbm.at[isrc_v], rows_v, sem).wait()
                pltpu.sync_copy(rows_v, acc_sh.at[idst_v], add=True)
                return carry

            lax.fori_loop(0, _NCH, chunk, 0)

            plsc.subcore_barrier()
            pltpu.sync_copy(acc_sh.at[pl.ds(s * rps, rps)],
                            out_hbm.at[c, pl.ds(off + s * rps, rps)])

    return k(rows, src, dstp0, dstp1)


def _tc_xw(x, W1):
    def body(x_ref, w_ref, o_ref):
        o_ref[...] = jnp.dot(x_ref[...].astype(jnp.bfloat16),
                             w_ref[...].astype(jnp.bfloat16),
                             preferred_element_type=jnp.float32)

    return pl.pallas_call(
        body,
        out_shape=jax.ShapeDtypeStruct((_N, _D), jnp.float32),
    )(x, W1)


def _tc_scale(degp, xw):
    def body(degp_ref, xw_ref, xws_ref, dsl_ref, di_ref):
        deg = degp_ref[0, :, 0:1] + degp_ref[1, :, 0:1]
        dsl = lax.rsqrt(deg + 1.0)
        di = jnp.where(deg > 0.0, lax.rsqrt(jnp.maximum(deg, 1.0)), 0.0)
        xws_ref[...] = xw_ref[...] * dsl
        dsl_ref[...] = dsl
        di_ref[...] = di

    return pl.pallas_call(
        body,
        out_shape=(
            jax.ShapeDtypeStruct((_N, _D), jnp.float32),
            jax.ShapeDtypeStruct((_N, 1), jnp.float32),
            jax.ShapeDtypeStruct((_N, 1), jnp.float32),
        ),
    )(degp, xw)


def _tc_h1gi(accp, xw, dsl, b1, W_ihT, b_ih):
    RB = 1000

    def body(ap_ref, xw_ref, dsl_ref, b1_ref, w_ref, bih_ref, gi_ref):
        acc = ap_ref[0] + ap_ref[1]
        dsl = dsl_ref[...]
        h1 = jnp.maximum(dsl * acc + (dsl * dsl) * xw_ref[...] + b1_ref[...],
                         0.0)
        gi_ref[...] = jnp.dot(h1.astype(jnp.bfloat16),
                              w_ref[...].astype(jnp.bfloat16),
                              preferred_element_type=jnp.float32) + bih_ref[...]

    return pl.pallas_call(
        body,
        grid=(_N // RB,),
        in_specs=[
            pl.BlockSpec((2, RB, _D), lambda i: (0, i, 0)),
            pl.BlockSpec((RB, _D), lambda i: (i, 0)),
            pl.BlockSpec((RB, 1), lambda i: (i, 0)),
            pl.BlockSpec((1, _D), lambda i: (0, 0)),
            pl.BlockSpec((_D, 3 * _H), lambda i: (0, 0)),
            pl.BlockSpec((1, 3 * _H), lambda i: (0, 0)),
        ],
        out_specs=pl.BlockSpec((RB, 3 * _H), lambda i: (i, 0)),
        out_shape=jax.ShapeDtypeStruct((_N, 3 * _H), jnp.float32),
    )(accp, xw, dsl, b1, W_ihT, b_ih)


def _tc_gru(GI, di, W_hhT, b_hh):
    SB = 80

    def body(gi_ref, di_ref, w_ref, bhh_ref, h_ref, hs_ref, hcar):
        wb = w_ref[...].astype(jnp.bfloat16)

        @pl.when(pl.program_id(0) == 0)
        def _():
            hcar[...] = jnp.zeros_like(hcar)

        def step(t, h):
            gi = gi_ref[pl.ds(t, 1), :]
            gh = jnp.dot(h.astype(jnp.bfloat16), wb,
                         preferred_element_type=jnp.float32) + bhh_ref[...]
            r = jax.nn.sigmoid(gi[:, 0:_H] + gh[:, 0:_H])
            z = jax.nn.sigmoid(gi[:, _H:2 * _H] + gh[:, _H:2 * _H])
            n = jnp.tanh(gi[:, 2 * _H:3 * _H] + r * gh[:, 2 * _H:3 * _H])
            hn = (1.0 - z) * n + z * h
            h_ref[pl.ds(t, 1), :] = hn
            hs_ref[pl.ds(t, 1), :] = hn * di_ref[pl.ds(t, 1), :]
            return hn

        hcar[...] = lax.fori_loop(0, SB, step, hcar[...])

    return pl.pallas_call(
        body,
        grid=(_N // SB,),
        in_specs=[
            pl.BlockSpec((SB, 3 * _H), lambda i: (i, 0)),
            pl.BlockSpec((SB, 1), lambda i: (i, 0)),
            pl.BlockSpec((_H, 3 * _H), lambda i: (0, 0)),
            pl.BlockSpec((1, 3 * _H), lambda i: (0, 0)),
        ],
        out_specs=(
            pl.BlockSpec((SB, _H), lambda i: (i, 0)),
            pl.BlockSpec((SB, _H), lambda i: (i, 0)),
        ),
        out_shape=(
            jax.ShapeDtypeStruct((_N, _H), jnp.float32),
            jax.ShapeDtypeStruct((_N, _H), jnp.float32),
        ),
        scratch_shapes=[pltpu.VMEM((1, _H), jnp.float32)],
    )(GI, di, W_hhT, b_hh)


def _tc_score(h, accp2, di, W_fc, b_fc):
    RB = 1000

    def body(h_ref, ap_ref, di_ref, wfc_ref, bfc_ref, sc_ref, lsm_ref):
        h = h_ref[...]
        agg = (ap_ref[0] + ap_ref[1]) * di_ref[...]
        sc_ref[...] = jnp.sum(jnp.abs(h - agg), axis=1, keepdims=True)
        logits = jnp.dot(h.astype(jnp.bfloat16),
                         wfc_ref[...].astype(jnp.bfloat16),
                         preferred_element_type=jnp.float32) + bfc_ref[...]
        m = jnp.max(logits, axis=1, keepdims=True)
        sh = logits - m
        lsm_ref[...] = sh - jnp.log(jnp.sum(jnp.exp(sh), axis=1,
                                            keepdims=True))

    return pl.pallas_call(
        body,
        grid=(_N // RB,),
        in_specs=[
            pl.BlockSpec((RB, _H), lambda i: (i, 0)),
            pl.BlockSpec((2, RB, _D), lambda i: (0, i, 0)),
            pl.BlockSpec((RB, 1), lambda i: (i, 0)),
            pl.BlockSpec((_H, 2), lambda i: (0, 0)),
            pl.BlockSpec((1, 2), lambda i: (0, 0)),
        ],
        out_specs=(
            pl.BlockSpec((RB, 1), lambda i: (i, 0)),
            pl.BlockSpec((RB, 2), lambda i: (i, 0)),
        ),
        out_shape=(
            jax.ShapeDtypeStruct((_N, 1), jnp.float32),
            jax.ShapeDtypeStruct((_N, 2), jnp.float32),
        ),
    )(h, accp2, di, W_fc, b_fc)


def _tc_sort(S, L0, L1):
    R, Cn = 128, 128

    def body(s_ref, a_ref, b_ref, oa_ref, ob_ref):
        S = s_ref[...]
        A = a_ref[...]
        B = b_ref[...]
        r = lax.broadcasted_iota(jnp.int32, (R, Cn), 0)
        c = lax.broadcasted_iota(jnp.int32, (R, Cn), 1)

        def lane_partner(x, j, first):
            down = jnp.concatenate([x[:, j:], x[:, :j]], axis=1)
            up = jnp.concatenate([x[:, Cn - j:], x[:, :Cn - j]], axis=1)
            return jnp.where(first, down, up)

        def row_partner(x, jr, first):
            down = jnp.concatenate([x[jr:, :], x[:jr, :]], axis=0)
            up = jnp.concatenate([x[R - jr:, :], x[:R - jr, :]], axis=0)
            return jnp.where(first, down, up)

        k = 2
        while k <= _SORTN:
            j = k // 2
            while j >= 1:
                if j < Cn:
                    first = (c & j) == 0
                    Sp = lane_partner(S, j, first)
                    Ap = lane_partner(A, j, first)
                    Bp = lane_partner(B, j, first)
                else:
                    jr = j // Cn
                    first = (r & jr) == 0
                    Sp = row_partner(S, jr, first)
                    Ap = row_partner(A, jr, first)
                    Bp = row_partner(B, jr, first)
                if k < Cn:
                    d = (c & k) == 0
                else:
                    d = (r & (k // Cn)) == 0
                lo = jnp.where(first, S, Sp)
                hi = jnp.where(first, Sp, S)
                swap = (d & (lo < hi)) | (jnp.logical_not(d) & (hi < lo))
                S = jnp.where(swap, Sp, S)
                A = jnp.where(swap, Ap, A)
                B = jnp.where(swap, Bp, B)
                j //= 2
            k *= 2

        oa_ref[...] = A
        ob_ref[...] = B

    return pl.pallas_call(
        body,
        out_shape=(
            jax.ShapeDtypeStruct((R, Cn), jnp.float32),
            jax.ShapeDtypeStruct((R, Cn), jnp.float32),
        ),
    )(S, L0, L1)


def kernel(x, edge_index, W1, b1, W_ih, W_hh, b_ih, b_hh, W_fc, b_fc):
    src = edge_index[0]
    dst = edge_index[1]
    dstp0 = jnp.where(dst < _NPH, dst, _NPH)
    dstp1 = jnp.where(dst >= _NPH, dst - _NPH, _NPH)

    degp = _sc_degree(dstp0, dstp1)
    xw = _tc_xw(x, W1)

    xws, dsl, di = _tc_scale(degp[:, :_N, :], xw)

    accp = _sc_scatter_rows(xws, src, dstp0, dstp1)

    gi = _tc_h1gi(accp[:, :_N, :], xw, dsl, b1.reshape(1, _D),
                  W_ih.T, b_ih.reshape(1, 3 * _H))

    h, hs = _tc_gru(gi, di, W_hh.T, b_hh.reshape(1, 3 * _H))

    accp2 = _sc_scatter_rows(hs, src, dstp0, dstp1)

    sc, lsm = _tc_score(h, accp2[:, :_N, :], di, W_fc,
                        b_fc.reshape(1, 2))

    pad = _SORTN - _N
    s_flat = jnp.concatenate([sc[:, 0], jnp.full((pad,), -1.0, jnp.float32)])
    l0 = jnp.concatenate([lsm[:, 0], jnp.zeros((pad,), jnp.float32)])
    l1 = jnp.concatenate([lsm[:, 1], jnp.zeros((pad,), jnp.float32)])

    o0, o1 = _tc_sort(s_flat.reshape(128, 128),
                      l0.reshape(128, 128), l1.reshape(128, 128))

    return jnp.stack([o0.reshape(-1)[:_K], o1.reshape(-1)[:_K]], axis=1)


# speedup vs baseline: 8.8366x; 1.1871x over previous
import functools

import jax
import jax.numpy as jnp
from jax import lax
from jax.experimental import pallas as pl
from jax.experimental.pallas import tpu as pltpu
from jax.experimental.pallas import tpu_sc as plsc

_N = 10000
_E = 320000
_D = 128
_H = 128
_K = 5000
_SORTN = 16384

_NCORES = 2
_NSUB = 16
_EPC = _E // _NCORES
_EPW = _E // (_NCORES * _NSUB)
_CH = 80
_NCH = _EPW // _CH
_NP = 10240
_NPH = _NP // 2

_HIGH = jax.lax.Precision.HIGHEST


def _sc_degree(dstp0, dstp1):
    mesh = plsc.VectorSubcoreMesh(core_axis_name="c", subcore_axis_name="s")
    rps = _NPH // _NSUB

    @functools.partial(
        pl.kernel,
        mesh=mesh,
        out_type=jax.ShapeDtypeStruct((_NCORES, _NP, _D), jnp.float32),
        scratch_types=[
            pltpu.VMEM((_CH,), jnp.int32),
            pltpu.VMEM((_CH, _D), jnp.float32),
            pltpu.VMEM((rps, _D), jnp.float32),
            pltpu.VMEM_SHARED((_NPH + 16, _D), jnp.float32),
        ],
    )
    def k(dstp0_hbm, dstp1_hbm, out_hbm, idx_v, ones_v, zbuf_v, acc_sh):
        c = lax.axis_index("c")
        s = lax.axis_index("s")
        zero16 = jnp.zeros((16,), jnp.float32)
        ones16 = jnp.ones((16,), jnp.float32)

        def fill_ones(i, carry):
            for j in range(_D // 16):
                ones_v[i, pl.ds(j * 16, 16)] = ones16
            return carry

        lax.fori_loop(0, _CH, fill_ones, 0)

        def fill_zero(i, carry):
            for j in range(_D // 16):
                zbuf_v[i, pl.ds(j * 16, 16)] = zero16
            return carry

        lax.fori_loop(0, rps, fill_zero, 0)

        ebase = c * _EPC + s * _EPW

        for off, dst_hbm in ((0, dstp0_hbm), (_NPH, dstp1_hbm)):
            pltpu.sync_copy(zbuf_v, acc_sh.at[pl.ds(s * rps, rps)])
            plsc.subcore_barrier()

            def chunk(i, carry):
                b = pl.multiple_of(ebase + i * _CH, 8)
                pltpu.sync_copy(dst_hbm.at[pl.ds(b, _CH)], idx_v)
                pltpu.sync_copy(ones_v, acc_sh.at[idx_v], add=True)
                return carry

            lax.fori_loop(0, _NCH, chunk, 0)

            plsc.subcore_barrier()
            pltpu.sync_copy(acc_sh.at[pl.ds(s * rps, rps)],
                            out_hbm.at[c, pl.ds(off + s * rps, rps)])

    return k(dstp0, dstp1)


def _sc_scatter_rows(rows, src, dstp0, dstp1):
    mesh = plsc.VectorSubcoreMesh(core_axis_name="c", subcore_axis_name="s")
    rps = _NPH // _NSUB

    @functools.partial(
        pl.kernel,
        mesh=mesh,
        out_type=jax.ShapeDtypeStruct((_NCORES, _NP, _D), jnp.float32),
        scratch_types=[
            pltpu.VMEM((_CH,), jnp.int32),
            pltpu.VMEM((_CH,), jnp.int32),
            pltpu.VMEM((_CH,), jnp.int32),
            pltpu.VMEM((_CH,), jnp.int32),
            pltpu.VMEM((_CH, _D), jnp.float32),
            pltpu.VMEM((_CH, _D), jnp.float32),
            pltpu.VMEM((rps, _D), jnp.float32),
            pltpu.VMEM_SHARED((_NPH + 16, _D), jnp.float32),
            pltpu.SemaphoreType.DMA,
            pltpu.SemaphoreType.DMA,
        ],
    )
    def k(rows_hbm, src_hbm, dstp0_hbm, dstp1_hbm, out_hbm,
          isrc0, isrc1, idst0, idst1, rows0, rows1, zbuf_v, acc_sh,
          sem0, sem1):
        c = lax.axis_index("c")
        s = lax.axis_index("s")
        zero16 = jnp.zeros((16,), jnp.float32)

        def fill_zero(i, carry):
            for j in range(_D // 16):
                zbuf_v[i, pl.ds(j * 16, 16)] = zero16
            return carry

        lax.fori_loop(0, rps, fill_zero, 0)

        ebase = c * _EPC + s * _EPW
        bufs = ((isrc0, idst0, rows0, sem0), (isrc1, idst1, rows1, sem1))

        for off, dst_hbm in ((0, dstp0_hbm), (_NPH, dstp1_hbm)):
            pltpu.sync_copy(zbuf_v, acc_sh.at[pl.ds(s * rps, rps)])
            plsc.subcore_barrier()

            for b, (isrc_b, idst_b, rows_b, sem_b) in enumerate(bufs):
                bb = pl.multiple_of(ebase + b * _CH, 8)
                pltpu.sync_copy(src_hbm.at[pl.ds(bb, _CH)], isrc_b)
                pltpu.sync_copy(dst_hbm.at[pl.ds(bb, _CH)], idst_b)
                pltpu.async_copy(rows_hbm.at[isrc_b], rows_b, sem_b)

            def giter(g2, carry):
                g = g2 * 2
                for b, (isrc_b, idst_b, rows_b, sem_b) in enumerate(bufs):
                    i = g + b

                    @pl.when(i < _NCH)
                    def _():
                        pltpu.make_async_copy(
                            rows_hbm.at[pl.ds(0, _CH)], rows_b, sem_b).wait()
                        pltpu.sync_copy(rows_b, acc_sh.at[idst_b], add=True)

                        @pl.when(i + 2 < _NCH)
                        def _():
                            b2 = pl.multiple_of(ebase + (i + 2) * _CH, 8)
                            pltpu.sync_copy(src_hbm.at[pl.ds(b2, _CH)], isrc_b)
                            pltpu.sync_copy(dst_hbm.at[pl.ds(b2, _CH)], idst_b)
                            pltpu.async_copy(rows_hbm.at[isrc_b], rows_b, sem_b)
                return carry

            lax.fori_loop(0, (_NCH + 1) // 2, giter, 0)

            plsc.subcore_barrier()
            pltpu.sync_copy(acc_sh.at[pl.ds(s * rps, rps)],
                            out_hbm.at[c, pl.ds(off + s * rps, rps)])

    return k(rows, src, dstp0, dstp1)


def _tc_xw(x, W1):
    def body(x_ref, w_ref, o_ref):
        o_ref[...] = jnp.dot(x_ref[...].astype(jnp.bfloat16),
                             w_ref[...].astype(jnp.bfloat16),
                             preferred_element_type=jnp.float32)

    return pl.pallas_call(
        body,
        out_shape=jax.ShapeDtypeStruct((_N, _D), jnp.float32),
    )(x, W1)


def _tc_scale(degp, xw):
    def body(degp_ref, xw_ref, xws_ref, dsl_ref, di_ref):
        deg = degp_ref[0, :, 0:1] + degp_ref[1, :, 0:1]
        dsl = lax.rsqrt(deg + 1.0)
        di = jnp.where(deg > 0.0, lax.rsqrt(jnp.maximum(deg, 1.0)), 0.0)
        xws_ref[...] = xw_ref[...] * dsl
        dsl_ref[...] = dsl
        di_ref[...] = di

    return pl.pallas_call(
        body,
        out_shape=(
            jax.ShapeDtypeStruct((_N, _D), jnp.float32),
            jax.ShapeDtypeStruct((_N, 1), jnp.float32),
            jax.ShapeDtypeStruct((_N, 1), jnp.float32),
        ),
    )(degp, xw)


def _tc_h1gi(accp, xw, dsl, b1, W_ihT, b_ih):
    RB = 1000

    def body(ap_ref, xw_ref, dsl_ref, b1_ref, w_ref, bih_ref, gi_ref):
        acc = ap_ref[0] + ap_ref[1]
        dsl = dsl_ref[...]
        h1 = jnp.maximum(dsl * acc + (dsl * dsl) * xw_ref[...] + b1_ref[...],
                         0.0)
        gi_ref[...] = jnp.dot(h1.astype(jnp.bfloat16),
                              w_ref[...].astype(jnp.bfloat16),
                              preferred_element_type=jnp.float32) + bih_ref[...]

    return pl.pallas_call(
        body,
        grid=(_N // RB,),
        in_specs=[
            pl.BlockSpec((2, RB, _D), lambda i: (0, i, 0)),
            pl.BlockSpec((RB, _D), lambda i: (i, 0)),
            pl.BlockSpec((RB, 1), lambda i: (i, 0)),
            pl.BlockSpec((1, _D), lambda i: (0, 0)),
            pl.BlockSpec((_D, 3 * _H), lambda i: (0, 0)),
            pl.BlockSpec((1, 3 * _H), lambda i: (0, 0)),
        ],
        out_specs=pl.BlockSpec((RB, 3 * _H), lambda i: (i, 0)),
        out_shape=jax.ShapeDtypeStruct((_N, 3 * _H), jnp.float32),
    )(accp, xw, dsl, b1, W_ihT, b_ih)


def _tc_gru(GI, di, W_hhT, b_hh):
    SB = 80

    def body(gi_ref, di_ref, w_ref, bhh_ref, h_ref, hs_ref, hcar):
        wb = w_ref[...].astype(jnp.bfloat16)

        @pl.when(pl.program_id(0) == 0)
        def _():
            hcar[...] = jnp.zeros_like(hcar)

        def step(t, h):
            gi = gi_ref[pl.ds(t, 1), :]
            gh = jnp.dot(h.astype(jnp.bfloat16), wb,
                         preferred_element_type=jnp.float32) + bhh_ref[...]
            r = jax.nn.sigmoid(gi[:, 0:_H] + gh[:, 0:_H])
            z = jax.nn.sigmoid(gi[:, _H:2 * _H] + gh[:, _H:2 * _H])
            n = jnp.tanh(gi[:, 2 * _H:3 * _H] + r * gh[:, 2 * _H:3 * _H])
            hn = (1.0 - z) * n + z * h
            h_ref[pl.ds(t, 1), :] = hn
            hs_ref[pl.ds(t, 1), :] = hn * di_ref[pl.ds(t, 1), :]
            return hn

        hcar[...] = lax.fori_loop(0, SB, step, hcar[...])

    return pl.pallas_call(
        body,
        grid=(_N // SB,),
        in_specs=[
            pl.BlockSpec((SB, 3 * _H), lambda i: (i, 0)),
            pl.BlockSpec((SB, 1), lambda i: (i, 0)),
            pl.BlockSpec((_H, 3 * _H), lambda i: (0, 0)),
            pl.BlockSpec((1, 3 * _H), lambda i: (0, 0)),
        ],
        out_specs=(
            pl.BlockSpec((SB, _H), lambda i: (i, 0)),
            pl.BlockSpec((SB, _H), lambda i: (i, 0)),
        ),
        out_shape=(
            jax.ShapeDtypeStruct((_N, _H), jnp.float32),
            jax.ShapeDtypeStruct((_N, _H), jnp.float32),
        ),
        scratch_shapes=[pltpu.VMEM((1, _H), jnp.float32)],
    )(GI, di, W_hhT, b_hh)


def _tc_score(h, accp2, di, W_fc, b_fc):
    RB = 1000

    def body(h_ref, ap_ref, di_ref, wfc_ref, bfc_ref, sc_ref, lsm_ref):
        h = h_ref[...]
        agg = (ap_ref[0] + ap_ref[1]) * di_ref[...]
        sc_ref[...] = jnp.sum(jnp.abs(h - agg), axis=1, keepdims=True)
        logits = jnp.dot(h.astype(jnp.bfloat16),
                         wfc_ref[...].astype(jnp.bfloat16),
                         preferred_element_type=jnp.float32) + bfc_ref[...]
        m = jnp.max(logits, axis=1, keepdims=True)
        sh = logits - m
        lsm_ref[...] = sh - jnp.log(jnp.sum(jnp.exp(sh), axis=1,
                                            keepdims=True))

    return pl.pallas_call(
        body,
        grid=(_N // RB,),
        in_specs=[
            pl.BlockSpec((RB, _H), lambda i: (i, 0)),
            pl.BlockSpec((2, RB, _D), lambda i: (0, i, 0)),
            pl.BlockSpec((RB, 1), lambda i: (i, 0)),
            pl.BlockSpec((_H, 2), lambda i: (0, 0)),
            pl.BlockSpec((1, 2), lambda i: (0, 0)),
        ],
        out_specs=(
            pl.BlockSpec((RB, 1), lambda i: (i, 0)),
            pl.BlockSpec((RB, 2), lambda i: (i, 0)),
        ),
        out_shape=(
            jax.ShapeDtypeStruct((_N, 1), jnp.float32),
            jax.ShapeDtypeStruct((_N, 2), jnp.float32),
        ),
    )(h, accp2, di, W_fc, b_fc)


def _tc_sort(S, L0, L1):
    R, Cn = 128, 128

    def body(s_ref, a_ref, b_ref, oa_ref, ob_ref):
        S = s_ref[...]
        A = a_ref[...]
        B = b_ref[...]
        r = lax.broadcasted_iota(jnp.int32, (R, Cn), 0)
        c = lax.broadcasted_iota(jnp.int32, (R, Cn), 1)

        def lane_partner(x, j, first):
            down = jnp.concatenate([x[:, j:], x[:, :j]], axis=1)
            up = jnp.concatenate([x[:, Cn - j:], x[:, :Cn - j]], axis=1)
            return jnp.where(first, down, up)

        def row_partner(x, jr, first):
            down = jnp.concatenate([x[jr:, :], x[:jr, :]], axis=0)
            up = jnp.concatenate([x[R - jr:, :], x[:R - jr, :]], axis=0)
            return jnp.where(first, down, up)

        k = 2
        while k <= _SORTN:
            j = k // 2
            while j >= 1:
                if j < Cn:
                    first = (c & j) == 0
                    Sp = lane_partner(S, j, first)
                    Ap = lane_partner(A, j, first)
                    Bp = lane_partner(B, j, first)
                else:
                    jr = j // Cn
                    first = (r & jr) == 0
                    Sp = row_partner(S, jr, first)
                    Ap = row_partner(A, jr, first)
                    Bp = row_partner(B, jr, first)
                if k < Cn:
                    d = (c & k) == 0
                else:
                    d = (r & (k // Cn)) == 0
                lo = jnp.where(first, S, Sp)
                hi = jnp.where(first, Sp, S)
                swap = (d & (lo < hi)) | (jnp.logical_not(d) & (hi < lo))
                S = jnp.where(swap, Sp, S)
                A = jnp.where(swap, Ap, A)
                B = jnp.where(swap, Bp, B)
                j //= 2
            k *= 2

        oa_ref[...] = A
        ob_ref[...] = B

    return pl.pallas_call(
        body,
        out_shape=(
            jax.ShapeDtypeStruct((R, Cn), jnp.float32),
            jax.ShapeDtypeStruct((R, Cn), jnp.float32),
        ),
    )(S, L0, L1)


def kernel(x, edge_index, W1, b1, W_ih, W_hh, b_ih, b_hh, W_fc, b_fc):
    src = edge_index[0]
    dst = edge_index[1]
    dstp0 = jnp.where(dst < _NPH, dst, _NPH)
    dstp1 = jnp.where(dst >= _NPH, dst - _NPH, _NPH)

    degp = _sc_degree(dstp0, dstp1)
    xw = _tc_xw(x, W1)

    xws, dsl, di = _tc_scale(degp[:, :_N, :], xw)

    accp = _sc_scatter_rows(xws, src, dstp0, dstp1)

    gi = _tc_h1gi(accp[:, :_N, :], xw, dsl, b1.reshape(1, _D),
                  W_ih.T, b_ih.reshape(1, 3 * _H))

    h, hs = _tc_gru(gi, di, W_hh.T, b_hh.reshape(1, 3 * _H))

    accp2 = _sc_scatter_rows(hs, src, dstp0, dstp1)

    sc, lsm = _tc_score(h, accp2[:, :_N, :], di, W_fc,
                        b_fc.reshape(1, 2))

    pad = _SORTN - _N
    s_flat = jnp.concatenate([sc[:, 0], jnp.full((pad,), -1.0, jnp.float32)])
    l0 = jnp.concatenate([lsm[:, 0], jnp.zeros((pad,), jnp.float32)])
    l1 = jnp.concatenate([lsm[:, 1], jnp.zeros((pad,), jnp.float32)])

    o0, o1 = _tc_sort(s_flat.reshape(128, 128),
                      l0.reshape(128, 128), l1.reshape(128, 128))

    return jnp.stack([o0.reshape(-1)[:_K], o1.reshape(-1)[:_K]], axis=1)
